# baseline (speedup 1.0000x reference)
import dataclasses
import functools

import jax
import jax.numpy as jnp
from jax import lax
from jax.experimental import pallas as pl
from jax.experimental.pallas import tpu as pltpu
from jax.experimental.pallas import tpu_sc as plsc

DIM = 64
BATCH = 16384
NUM_CORES = 2
NUM_SUBCORES = 16
NUM_WORKERS = NUM_CORES * NUM_SUBCORES
ROWS_PER_WORKER = BATCH // NUM_WORKERS
GATHER_CHUNK = 128
NUM_GATHERS = ROWS_PER_WORKER // GATHER_CHUNK
LANES = 16
DIM_CHUNKS = DIM // LANES


def _sc_kernel(diag_hbm, idx_hbm, h_hbm, t_hbm, out_hbm,
               idx_v, rel_v, h_v, t_v, out_v, sem_g, sem_h, sem_t):
  wid = lax.axis_index("s") * NUM_CORES + lax.axis_index("c")
  base = wid * ROWS_PER_WORKER

  pltpu.sync_copy(idx_hbm.at[wid], idx_v)
  gathers = [
      pltpu.async_copy(
          diag_hbm.at[idx_v.at[j]],
          rel_v.at[pl.ds(j * GATHER_CHUNK, GATHER_CHUNK)],
          sem_g,
      )
      for j in range(NUM_GATHERS)
  ]
  copy_h = pltpu.async_copy(h_hbm.at[pl.ds(base, ROWS_PER_WORKER)], h_v, sem_h)
  copy_t = pltpu.async_copy(t_hbm.at[pl.ds(base, ROWS_PER_WORKER)], t_v, sem_t)
  for g in gathers:
    g.wait()
  copy_h.wait()
  copy_t.wait()

  lane = lax.iota(jnp.int32, LANES)

  @pl.loop(0, ROWS_PER_WORKER // LANES)
  def _(g):
    res = jnp.zeros((LANES,), jnp.float32)
    for k in range(LANES):
      i = g * LANES + k
      acc = (h_v[i, pl.ds(0, LANES)] * t_v[i, pl.ds(0, LANES)]
             * rel_v[i, pl.ds(0, LANES)])
      for c in range(1, DIM_CHUNKS):
        sl = pl.ds(c * LANES, LANES)
        acc = acc + h_v[i, sl] * t_v[i, sl] * rel_v[i, sl]
      res = jnp.where(lane == k, jnp.sum(acc), res)
    out_v[pl.ds(g * LANES, LANES)] = res

  pltpu.sync_copy(out_v, out_hbm.at[pl.ds(base, ROWS_PER_WORKER)])


@jax.jit
def _dist_mult(h, r, t, diag):
  idx = r.astype(jnp.int32).reshape(NUM_WORKERS, NUM_GATHERS, GATHER_CHUNK)
  mesh = plsc.VectorSubcoreMesh(core_axis_name="c", subcore_axis_name="s")
  cp = pltpu.CompilerParams()
  for field, value in (("needs_layout_passes", False),
                       ("use_tc_tiling_on_sc", False)):
    if field in pltpu.CompilerParams.__dataclass_fields__:
      cp = dataclasses.replace(cp, **{field: value})
  run = pl.kernel(
      _sc_kernel,
      out_type=jax.ShapeDtypeStruct((BATCH,), jnp.float32),
      mesh=mesh,
      compiler_params=cp,
      scratch_types=[
          pltpu.VMEM((NUM_GATHERS, GATHER_CHUNK), jnp.int32),
          pltpu.VMEM((ROWS_PER_WORKER, DIM), jnp.float32),
          pltpu.VMEM((ROWS_PER_WORKER, DIM), jnp.float32),
          pltpu.VMEM((ROWS_PER_WORKER, DIM), jnp.float32),
          pltpu.VMEM((ROWS_PER_WORKER,), jnp.float32),
          pltpu.SemaphoreType.DMA,
          pltpu.SemaphoreType.DMA,
          pltpu.SemaphoreType.DMA,
      ],
  )
  return run(diag, idx, h, t)


def kernel(h, r, t, diag):
  return _dist_mult(h, r, t, diag)
